# baseline (speedup 1.0000x reference)
import jax
import jax.numpy as jnp
from jax import lax
from jax.experimental import pallas as pl
from jax.experimental.pallas import tpu as pltpu

B = 16384
D = 1024
C = 1000
CP = 1024
M = 2 * CP
NPAD = 2 * (CP - C)

RB1 = 2048
NB1 = B // RB1
RB2 = 1024
NB2 = B // RB2

DE = D + 128


def _stats_body(feat_ref, lbl_ref, mem_ref, src_ref,
                featn_ref, memot_ref, lltot_ref, sums_ref):
    i = pl.program_id(0)

    x = feat_ref[...]
    ss = jnp.sum(x * x, axis=1, keepdims=True)
    inv = 1.0 / jnp.maximum(jnp.sqrt(ss), 1e-12)
    xn = x * inv
    xnb = xn.astype(jnp.bfloat16)
    featn_ref[...] = xnb

    lbl = lbl_ref[0, 0, :]
    cls = lax.broadcasted_iota(jnp.int32, (CP, RB1), 0)
    oh = (cls == lbl[None, :]).astype(jnp.bfloat16)

    rhs = jnp.concatenate(
        [xnb, jnp.ones((RB1, 128), jnp.bfloat16)], axis=1)

    @pl.when(i == 0)
    def _():
        sums_ref[...] = jnp.zeros_like(sums_ref)

    sums_ref[...] += lax.dot_general(
        oh, rhs, (((1,), (0,)), ((), ())),
        preferred_element_type=jnp.float32)

    @pl.when(i == NB1 - 1)
    def _():
        sums = sums_ref[:, 0:D]
        counts = jnp.max(sums_ref[:, D:DE], axis=1, keepdims=True)
        present = counts > 0.0
        snorm = jnp.sqrt(jnp.sum(sums * sums, axis=1, keepdims=True))
        bc = jnp.where(present, sums / jnp.maximum(snorm, 1e-12), 0.0)
        mem = mem_ref[...]
        uw = jnp.sum(mem * bc, axis=1, keepdims=True)
        uw = 1.0 - (1.0 - uw) * present.astype(jnp.float32)
        nm = uw * mem + (1.0 - uw) * bc
        nnorm = jnp.sqrt(jnp.sum(nm * nm, axis=1, keepdims=True))
        nm = nm / jnp.maximum(nnorm, 1e-12)
        memot_ref[:, 0:CP] = jnp.transpose(nm).astype(jnp.bfloat16)
        memot_ref[:, CP:M] = jnp.transpose(src_ref[...]).astype(jnp.bfloat16)
        lltot_ref[...] = jnp.sum(sums * nm).reshape(1, 1)


def _loss_body(featn_ref, memot_ref, lltot_ref, out_ref, acc_ref):
    i = pl.program_id(0)
    x = featn_ref[...]
    logits = lax.dot_general(
        x, memot_ref[...],
        (((1,), (0,)), ((), ())), preferred_element_type=jnp.float32)
    es = jnp.sum(jnp.exp(logits), axis=1, keepdims=True) - float(NPAD)
    lse = jnp.log(es)

    @pl.when(i == 0)
    def _():
        acc_ref[...] = jnp.zeros_like(acc_ref)

    acc_ref[...] += lse.reshape(8, RB2 // 8)

    @pl.when(i == NB2 - 1)
    def _():
        out_ref[...] = (jnp.sum(acc_ref[...]).reshape(1, 1)
                        - lltot_ref[...]) / float(B)


@jax.jit
def kernel(feat, label, memory, source_memo):
    lbl3 = label.astype(jnp.int32).reshape(NB1, 1, RB1)
    mem_p = jnp.pad(memory, ((0, CP - C), (0, 0)))
    src_p = jnp.pad(source_memo, ((0, CP - C), (0, 0)))

    featn, memot, lltot = pl.pallas_call(
        _stats_body,
        grid=(NB1,),
        in_specs=[
            pl.BlockSpec((RB1, D), lambda i: (i, 0)),
            pl.BlockSpec((1, 1, RB1), lambda i: (i, 0, 0)),
            pl.BlockSpec((CP, D), lambda i: (0, 0)),
            pl.BlockSpec((CP, D), lambda i: (0, 0)),
        ],
        out_specs=[
            pl.BlockSpec((RB1, D), lambda i: (i, 0)),
            pl.BlockSpec((D, M), lambda i: (0, 0)),
            pl.BlockSpec((1, 1), lambda i: (0, 0)),
        ],
        out_shape=[
            jax.ShapeDtypeStruct((B, D), jnp.bfloat16),
            jax.ShapeDtypeStruct((D, M), jnp.bfloat16),
            jax.ShapeDtypeStruct((1, 1), jnp.float32),
        ],
        scratch_shapes=[
            pltpu.VMEM((CP, DE), jnp.float32),
        ],
        compiler_params=pltpu.CompilerParams(
            dimension_semantics=("arbitrary",)),
    )(feat, lbl3, mem_p, src_p)

    loss2d = pl.pallas_call(
        _loss_body,
        grid=(NB2,),
        in_specs=[
            pl.BlockSpec((RB2, D), lambda i: (i, 0)),
            pl.BlockSpec((D, M), lambda i: (0, 0)),
            pl.BlockSpec((1, 1), lambda i: (0, 0)),
        ],
        out_specs=pl.BlockSpec((1, 1), lambda i: (0, 0)),
        out_shape=jax.ShapeDtypeStruct((1, 1), jnp.float32),
        scratch_shapes=[pltpu.VMEM((8, RB2 // 8), jnp.float32)],
        compiler_params=pltpu.CompilerParams(
            dimension_semantics=("arbitrary",)),
    )(featn, memot, lltot)

    return loss2d[0, 0]
